# Initial kernel scaffold; baseline (speedup 1.0000x reference)
import jax
import jax.numpy as jnp
from jax.experimental import pallas as pl


def kernel(theta_t, pos_t, t, batch, node_w, node_b, te_w1, te_b1, te_w2, te_b2, tp_w, tp_b, dist_w1, dist_b1, dist_w2, dist_b2, dir_w1, dir_b1, dir_w2, dir_b2, en_g, en_b, msg_w1, msg_b1, msg_w2, msg_b2, ln1_g, ln1_b, ffn_w1, ffn_b1, ffn_w2, ffn_b2, ln2_g, ln2_b, pp_w, pp_b):
    raise NotImplementedError("write your pallas kernel here")


# speedup vs baseline: 2.6143x; 2.6143x over previous
import jax
import jax.numpy as jnp
from jax.experimental import pallas as pl

N = 10000
B = 16
HIDDEN = 128
NUM_LAYERS = 4
TIME_EMB = 128
CUTOFF = 5.0
K = 32

_RC = 200
_EC = 200
_MC = 2000


def _radius_kernel(pos_c_ref, pos_t_ref, bat_c_ref, bat_r_ref, col_ref, msk_ref):
    i = pl.program_id(0)
    pos_c = pos_c_ref[...]
    pos_t = pos_t_ref[...]
    sq_c = jnp.sum(pos_c * pos_c, axis=1, keepdims=True)
    sq_r = jnp.sum(pos_t * pos_t, axis=0, keepdims=True)
    d2 = sq_c + sq_r - 2.0 * jnp.dot(pos_c, pos_t, preferred_element_type=jnp.float32)
    col_ids = jax.lax.broadcasted_iota(jnp.int32, (_RC, N), 1)
    row_ids = i * _RC + jax.lax.broadcasted_iota(jnp.int32, (_RC, N), 0)
    valid = (bat_c_ref[...] == bat_r_ref[...]) & (d2 <= CUTOFF * CUTOFF) & (col_ids != row_ids)
    d2m = jnp.where(valid, d2, jnp.inf)
    k_ids = jax.lax.broadcasted_iota(jnp.int32, (_RC, K), 1)

    def body(k, carry):
        dm, colacc, mskacc = carry
        mn = jnp.min(dm, axis=1, keepdims=True)
        sel = dm == mn
        idx = jnp.min(jnp.where(sel, col_ids, N), axis=1, keepdims=True)
        kk = k_ids == k
        colacc = jnp.where(kk, idx, colacc)
        mskacc = jnp.where(kk, (mn < jnp.inf).astype(jnp.float32), mskacc)
        dm = jnp.where(col_ids == idx, jnp.inf, dm)
        return dm, colacc, mskacc

    init = (d2m, jnp.zeros((_RC, K), jnp.int32), jnp.zeros((_RC, K), jnp.float32))
    _, cols, msks = jax.lax.fori_loop(0, K, body, init)
    col_ref[...] = cols
    msk_ref[...] = msks


def _edgefeat_kernel(ev_ref, dw1_ref, db1_ref, dw2_ref, db2_ref,
                     rw1_ref, rb1_ref, rw2_ref, rb2_ref, g_ref, b_ref, out_ref):
    ev = ev_ref[...].reshape(_EC * K, 3)
    dist = jnp.sqrt(jnp.sum(ev * ev, axis=1, keepdims=True))
    dirv = ev / (dist + 1e-8)
    df = jax.nn.silu(dist * dw1_ref[...] + db1_ref[...])
    df = jnp.dot(df, dw2_ref[...], preferred_element_type=jnp.float32) + db2_ref[...]
    rf = jax.nn.silu(jnp.dot(dirv, rw1_ref[...], preferred_element_type=jnp.float32) + rb1_ref[...])
    rf = jnp.dot(rf, rw2_ref[...], preferred_element_type=jnp.float32) + rb2_ref[...]
    ef = jnp.concatenate([df, rf], axis=1)
    mu = jnp.mean(ef, axis=1, keepdims=True)
    var = jnp.mean((ef - mu) ** 2, axis=1, keepdims=True)
    ef = (ef - mu) * jax.lax.rsqrt(var + 1e-5) * g_ref[...] + b_ref[...]
    out_ref[...] = ef.reshape(_EC, K, HIDDEN)


def _matmul_kernel(x_ref, w_ref, o_ref):
    o_ref[...] = jnp.dot(x_ref[...], w_ref[...], preferred_element_type=jnp.float32)


def _init_kernel(th_ref, tg_ref, flag_ref, w_ref, b_ref, o_ref):
    th = th_ref[...]
    mx = jnp.max(th, axis=1, keepdims=True)
    e = jnp.exp(th - mx)
    sm = e / jnp.sum(e, axis=1, keepdims=True)
    xin = jnp.where(flag_ref[0, 0] > 0, sm, th)
    o_ref[...] = (jnp.dot(xin, w_ref[...], preferred_element_type=jnp.float32)
                  + b_ref[...] + tg_ref[...])


def _layer_kernel(x_ref, bn_ref, ef_ref, mk_ref, wa_ref, wc_ref, b1_ref,
                  w2_ref, b2_ref, l1g_ref, l1b_ref, f1_ref, fb1_ref,
                  f2_ref, fb2_ref, l2g_ref, l2b_ref, out_ref):
    x = x_ref[...]
    a = jnp.dot(x, wa_ref[...], preferred_element_type=jnp.float32)
    ef = ef_ref[...].reshape(_EC * K, HIDDEN)
    c = jnp.dot(ef, wc_ref[...], preferred_element_type=jnp.float32)
    pre = a[:, None, :] + bn_ref[...] + c.reshape(_EC, K, HIDDEN) + b1_ref[...][None]
    h = jax.nn.silu(pre).reshape(_EC * K, HIDDEN)
    h = jnp.dot(h, w2_ref[...], preferred_element_type=jnp.float32) + b2_ref[...]
    m = jnp.sum(h.reshape(_EC, K, HIDDEN) * mk_ref[...][:, :, None], axis=1)
    x1 = x + m
    mu = jnp.mean(x1, axis=1, keepdims=True)
    var = jnp.mean((x1 - mu) ** 2, axis=1, keepdims=True)
    x1 = (x1 - mu) * jax.lax.rsqrt(var + 1e-5) * l1g_ref[...] + l1b_ref[...]
    f = jax.nn.silu(jnp.dot(x1, f1_ref[...], preferred_element_type=jnp.float32) + fb1_ref[...])
    f = jnp.dot(f, f2_ref[...], preferred_element_type=jnp.float32) + fb2_ref[...]
    x2 = x1 + f
    mu = jnp.mean(x2, axis=1, keepdims=True)
    var = jnp.mean((x2 - mu) ** 2, axis=1, keepdims=True)
    out_ref[...] = (x2 - mu) * jax.lax.rsqrt(var + 1e-5) * l2g_ref[...] + l2b_ref[...]


def _pool_kernel(x_ref, br_ref, bc_ref, w_ref, b_ref, o_ref):
    x = x_ref[...]
    br = br_ref[...]
    bc = bc_ref[...]
    biota = jax.lax.broadcasted_iota(jnp.int32, (B, N), 0)
    masks = (br == biota).astype(jnp.float32)
    counts = jnp.sum(masks, axis=1, keepdims=True)
    sums = jnp.dot(masks, x, preferred_element_type=jnp.float32)
    mean = sums / jnp.maximum(counts, 1.0)
    maxs = []
    for bi in range(B):
        mb = jnp.max(jnp.where(bc == bi, x, -jnp.inf), axis=0, keepdims=True)
        maxs.append(mb)
    mx = jnp.concatenate(maxs, axis=0)
    mx = jnp.where(counts > 0, mx, 0.0)
    gf = jnp.concatenate([mean, mx], axis=1)
    o_ref[...] = jax.nn.silu(jnp.dot(gf, w_ref[...], preferred_element_type=jnp.float32) + b_ref[...])


def _full(shape):
    return pl.BlockSpec(shape, lambda i: tuple(0 for _ in shape))


def kernel(theta_t, pos_t, t, batch, node_w, node_b, te_w1, te_b1, te_w2, te_b2,
           tp_w, tp_b, dist_w1, dist_b1, dist_w2, dist_b2, dir_w1, dir_b1,
           dir_w2, dir_b2, en_g, en_b, msg_w1, msg_b1, msg_w2, msg_b2,
           ln1_g, ln1_b, ffn_w1, ffn_b1, ffn_w2, ffn_b2, ln2_g, ln2_b, pp_w, pp_b):
    batch = batch.astype(jnp.int32)
    bcol = batch.reshape(N, 1)
    brow = batch.reshape(1, N)

    col, msk = pl.pallas_call(
        _radius_kernel,
        grid=(N // _RC,),
        in_specs=[
            pl.BlockSpec((_RC, 3), lambda i: (i, 0)),
            _full((3, N)),
            pl.BlockSpec((_RC, 1), lambda i: (i, 0)),
            _full((1, N)),
        ],
        out_specs=[
            pl.BlockSpec((_RC, K), lambda i: (i, 0)),
            pl.BlockSpec((_RC, K), lambda i: (i, 0)),
        ],
        out_shape=[
            jax.ShapeDtypeStruct((N, K), jnp.int32),
            jax.ShapeDtypeStruct((N, K), jnp.float32),
        ],
    )(pos_t, pos_t.T, bcol, brow)

    nbr_pos = jnp.take(pos_t, col.reshape(-1), axis=0).reshape(N, K, 3)
    edge_vec = nbr_pos - pos_t[:, None, :]
    ef = pl.pallas_call(
        _edgefeat_kernel,
        grid=(N // _EC,),
        in_specs=[
            pl.BlockSpec((_EC, K, 3), lambda i: (i, 0, 0)),
            _full((1, 64)), _full((1, 64)), _full((64, 64)), _full((1, 64)),
            _full((3, 64)), _full((1, 64)), _full((64, 64)), _full((1, 64)),
            _full((1, HIDDEN)), _full((1, HIDDEN)),
        ],
        out_specs=pl.BlockSpec((_EC, K, HIDDEN), lambda i: (i, 0, 0)),
        out_shape=jax.ShapeDtypeStruct((N, K, HIDDEN), jnp.float32),
    )(edge_vec, dist_w1.reshape(1, 64), dist_b1.reshape(1, 64), dist_w2,
      dist_b2.reshape(1, 64), dir_w1, dir_b1.reshape(1, 64), dir_w2,
      dir_b2.reshape(1, 64), en_g.reshape(1, HIDDEN), en_b.reshape(1, HIDDEN))

    half = TIME_EMB // 2
    inv_freq = 1.0 / (10000.0 ** (jnp.arange(half, dtype=jnp.float32) / half))
    si = t[:, None] * inv_freq[None, :]
    emb = jnp.concatenate([jnp.sin(si), jnp.cos(si)], axis=-1)
    emb = jax.nn.silu(emb @ te_w1 + te_b1) @ te_w2 + te_b2
    t_emb = emb @ tp_w + tp_b
    tg = jnp.take(t_emb, batch, axis=0)
    flag = ((jnp.min(theta_t) < 0) | (jnp.max(theta_t) > 1.0)).astype(jnp.float32).reshape(1, 1)

    x = pl.pallas_call(
        _init_kernel,
        grid=(N // _MC,),
        in_specs=[
            pl.BlockSpec((_MC, theta_t.shape[1]), lambda i: (i, 0)),
            pl.BlockSpec((_MC, HIDDEN), lambda i: (i, 0)),
            _full((1, 1)),
            _full((theta_t.shape[1], HIDDEN)),
            _full((1, HIDDEN)),
        ],
        out_specs=pl.BlockSpec((_MC, HIDDEN), lambda i: (i, 0)),
        out_shape=jax.ShapeDtypeStruct((N, HIDDEN), jnp.float32),
    )(theta_t, tg, flag, node_w, node_b.reshape(1, HIDDEN))

    flat_col = col.reshape(-1)
    for l in range(NUM_LAYERS):
        wa = msg_w1[l, :HIDDEN]
        wb = msg_w1[l, HIDDEN:2 * HIDDEN]
        wc = msg_w1[l, 2 * HIDDEN:]
        ball = pl.pallas_call(
            _matmul_kernel,
            grid=(N // _MC,),
            in_specs=[pl.BlockSpec((_MC, HIDDEN), lambda i: (i, 0)),
                      _full((HIDDEN, HIDDEN))],
            out_specs=pl.BlockSpec((_MC, HIDDEN), lambda i: (i, 0)),
            out_shape=jax.ShapeDtypeStruct((N, HIDDEN), jnp.float32),
        )(x, wb)
        bn = jnp.take(ball, flat_col, axis=0).reshape(N, K, HIDDEN)
        x = pl.pallas_call(
            _layer_kernel,
            grid=(N // _EC,),
            in_specs=[
                pl.BlockSpec((_EC, HIDDEN), lambda i: (i, 0)),
                pl.BlockSpec((_EC, K, HIDDEN), lambda i: (i, 0, 0)),
                pl.BlockSpec((_EC, K, HIDDEN), lambda i: (i, 0, 0)),
                pl.BlockSpec((_EC, K), lambda i: (i, 0)),
                _full((HIDDEN, HIDDEN)), _full((HIDDEN, HIDDEN)),
                _full((1, HIDDEN)), _full((HIDDEN, HIDDEN)), _full((1, HIDDEN)),
                _full((1, HIDDEN)), _full((1, HIDDEN)),
                _full((HIDDEN, 2 * HIDDEN)), _full((1, 2 * HIDDEN)),
                _full((2 * HIDDEN, HIDDEN)), _full((1, HIDDEN)),
                _full((1, HIDDEN)), _full((1, HIDDEN)),
            ],
            out_specs=pl.BlockSpec((_EC, HIDDEN), lambda i: (i, 0)),
            out_shape=jax.ShapeDtypeStruct((N, HIDDEN), jnp.float32),
        )(x, bn, ef, msk, wa, wc, msg_b1[l].reshape(1, HIDDEN), msg_w2[l],
          msg_b2[l].reshape(1, HIDDEN), ln1_g[l].reshape(1, HIDDEN),
          ln1_b[l].reshape(1, HIDDEN), ffn_w1[l], ffn_b1[l].reshape(1, 2 * HIDDEN),
          ffn_w2[l], ffn_b2[l].reshape(1, HIDDEN), ln2_g[l].reshape(1, HIDDEN),
          ln2_b[l].reshape(1, HIDDEN))

    out = pl.pallas_call(
        _pool_kernel,
        grid=(1,),
        in_specs=[
            _full((N, HIDDEN)), _full((1, N)), _full((N, 1)),
            _full((2 * HIDDEN, HIDDEN)), _full((1, HIDDEN)),
        ],
        out_specs=_full((B, HIDDEN)),
        out_shape=jax.ShapeDtypeStruct((B, HIDDEN), jnp.float32),
    )(x, brow, bcol, pp_w, pp_b.reshape(1, HIDDEN))
    return out
